# Initial kernel scaffold; baseline (speedup 1.0000x reference)
import jax
import jax.numpy as jnp
from jax.experimental import pallas as pl


def kernel(particles, edges):
    raise NotImplementedError("write your pallas kernel here")


# speedup vs baseline: 10.4083x; 10.4083x over previous
import functools

import jax
import jax.numpy as jnp
from jax import lax
from jax.experimental import pallas as pl
from jax.experimental.pallas import tpu as pltpu
from jax.experimental.pallas import tpu_sc as plsc

_N_NODES = 100000
_E = 1600000
_NC = 2
_NS = 16
_NW = _NC * _NS
_CHUNK = 128
_EPW = -(-(_E // _NW) // _CHUNK) * _CHUNK
_E_PAD = _EPW * _NW


def _rsqrt(x):
    i = plsc.bitcast(x, jnp.int32)
    i = jnp.int32(0x5F3759DF) - (i >> 1)
    y = plsc.bitcast(i, jnp.float32)
    xh = x * 0.5
    for _ in range(3):
        y = y * (1.5 - xh * y * y)
    return y


def _qblock(qi, qj, eidx, ob):
    for p in range(4):
        aw, ax, ay, az = qi[4 * p:4 * p + 4]
        bw, bx, by, bz = qj[4 * p:4 * p + 4]
        w = aw * bw + ax * bx + ay * by + az * bz
        x = ax * bw - aw * bx - ay * bz + az * by
        y = ay * bw - aw * by + ax * bz - az * bx
        z = az * bw - aw * bz - ax * by + ay * bx
        r = _rsqrt(w * w + x * x + y * y + z * z)
        for c, v in enumerate((w * r, x * r, y * r, z * r)):
            col = jnp.full((16,), 4 * p + c, jnp.int32)
            plsc.store_scatter(ob, [eidx, col], v)


def _sc_body(table_hbm, ei_hbm, ej_hbm, out_hbm,
             idx_i, idx_j, xi_b, xj_b, ob, sem):
    wid = lax.axis_index("s") * _NC + lax.axis_index("c")
    wbase = wid * _EPW

    def chunk_body(cix, carry):
        base = wbase + cix * _CHUNK
        pltpu.sync_copy(ei_hbm.at[pl.ds(base, _CHUNK)], idx_i)
        pltpu.sync_copy(ej_hbm.at[pl.ds(base, _CHUNK)], idx_j)
        cp1 = pltpu.async_copy(table_hbm.at[idx_i], xi_b, sem)
        cp2 = pltpu.async_copy(table_hbm.at[idx_j], xj_b, sem)
        cp1.wait()
        cp2.wait()
        for b in range(_CHUNK // 16):
            rows = lax.iota(jnp.int32, 16) + (b * 16)
            qi = [plsc.load_gather(xi_b, [rows, jnp.full((16,), c, jnp.int32)])
                  for c in range(16)]
            qj = [plsc.load_gather(xj_b, [rows, jnp.full((16,), c, jnp.int32)])
                  for c in range(16)]
            _qblock(qi, qj, rows, ob)
        pltpu.sync_copy(ob, out_hbm.at[pl.ds(base, _CHUNK)])
        return carry

    lax.fori_loop(0, _EPW // _CHUNK, chunk_body, 0)


def kernel(particles, edges):
    table = particles.reshape(_N_NODES, 16)
    pad = _E_PAD - _E
    ei = jnp.concatenate([edges[0], jnp.zeros((pad,), jnp.int32)])
    ej = jnp.concatenate([edges[1], jnp.zeros((pad,), jnp.int32)])

    mesh = plsc.VectorSubcoreMesh(core_axis_name="c", subcore_axis_name="s")
    run = functools.partial(
        pl.kernel,
        mesh=mesh,
        compiler_params=pltpu.CompilerParams(
            use_tc_tiling_on_sc=False, needs_layout_passes=False),
        out_type=jax.ShapeDtypeStruct((_E_PAD, 16), jnp.float32),
        scratch_types=[
            pltpu.VMEM((_CHUNK,), jnp.int32),
            pltpu.VMEM((_CHUNK,), jnp.int32),
            pltpu.VMEM((_CHUNK, 16), jnp.float32),
            pltpu.VMEM((_CHUNK, 16), jnp.float32),
            pltpu.VMEM((_CHUNK, 16), jnp.float32),
            pltpu.SemaphoreType.DMA,
        ],
    )(_sc_body)
    out = run(table, ei, ej)
    return out[:_E].reshape(_E, 4, 4)


# speedup vs baseline: 12.8503x; 1.2346x over previous
import functools

import jax
import jax.numpy as jnp
from jax import lax
from jax.experimental import pallas as pl
from jax.experimental.pallas import tpu as pltpu
from jax.experimental.pallas import tpu_sc as plsc

_N_NODES = 100000
_E = 1600000
_NC = 2
_NS = 16
_NW = _NC * _NS
_CHUNK = 256
_EPW = -(-(_E // _NW) // (2 * _CHUNK)) * (2 * _CHUNK)
_E_PAD = _EPW * _NW
_NCHUNKS = _EPW // _CHUNK


def _rsqrt(x):
    i = plsc.bitcast(x, jnp.int32)
    i = jnp.int32(0x5F3759DF) - (i >> 1)
    y = plsc.bitcast(i, jnp.float32)
    xh = x * 0.5
    for _ in range(3):
        y = y * (1.5 - xh * y * y)
    return y


def _compute_chunk(xi_b, xj_b, ob):
    def block(b, carry):
        rows = lax.iota(jnp.int32, 16) + b * 16
        qi = [plsc.load_gather(xi_b, [rows, jnp.full((16,), c, jnp.int32)])
              for c in range(16)]
        qj = [plsc.load_gather(xj_b, [rows, jnp.full((16,), c, jnp.int32)])
              for c in range(16)]
        for p in range(4):
            aw, ax, ay, az = qi[4 * p:4 * p + 4]
            bw, bx, by, bz = qj[4 * p:4 * p + 4]
            w = aw * bw + ax * bx + ay * by + az * bz
            x = ax * bw - aw * bx - ay * bz + az * by
            y = ay * bw - aw * by + ax * bz - az * bx
            z = az * bw - aw * bz - ax * by + ay * bx
            r = _rsqrt(w * w + x * x + y * y + z * z)
            for c, v in enumerate((w * r, x * r, y * r, z * r)):
                col = jnp.full((16,), 4 * p + c, jnp.int32)
                plsc.store_scatter(ob, [rows, col], v)
        return carry

    lax.fori_loop(0, _CHUNK // 16, block, 0, unroll=4)


def _sc_body(table_hbm, ei_hbm, ej_hbm, out_hbm,
             ii0, ij0, ii1, ij1, xi0, xj0, xi1, xj1, ob,
             sg0, sg1):
    wid = lax.axis_index("s") * _NC + lax.axis_index("c")
    wbase = wid * _EPW
    n = _NCHUNKS

    def idx_load(slot_ii, slot_ij, base):
        pltpu.sync_copy(ei_hbm.at[pl.ds(base, _CHUNK)], slot_ii)
        pltpu.sync_copy(ej_hbm.at[pl.ds(base, _CHUNK)], slot_ij)

    def gather_issue(slot_ii, slot_ij, xi_b, xj_b, sem):
        pltpu.async_copy(table_hbm.at[slot_ii], xi_b, sem)
        pltpu.async_copy(table_hbm.at[slot_ij], xj_b, sem)

    def gather_wait(slot_ii, slot_ij, xi_b, xj_b, sem):
        pltpu.make_async_copy(table_hbm.at[slot_ii], xi_b, sem).wait()
        pltpu.make_async_copy(table_hbm.at[slot_ij], xj_b, sem).wait()

    idx_load(ii0, ij0, wbase)
    gather_issue(ii0, ij0, xi0, xj0, sg0)

    def half_iter(g, cur, nxt):
        (ii_c, ij_c, xi_c, xj_c, sg_c) = cur
        (ii_n, ij_n, xi_n, xj_n, sg_n) = nxt
        base_g = wbase + g * _CHUNK
        base_n = wbase + jnp.minimum(g + 1, n - 1) * _CHUNK
        idx_load(ii_n, ij_n, base_n)
        gather_issue(ii_n, ij_n, xi_n, xj_n, sg_n)
        gather_wait(ii_c, ij_c, xi_c, xj_c, sg_c)
        _compute_chunk(xi_c, xj_c, ob)
        pltpu.sync_copy(ob, out_hbm.at[pl.ds(base_g, _CHUNK)])

    slot0 = (ii0, ij0, xi0, xj0, sg0)
    slot1 = (ii1, ij1, xi1, xj1, sg1)

    def loop_body(t, carry):
        half_iter(2 * t, slot0, slot1)
        half_iter(2 * t + 1, slot1, slot0)
        return carry

    lax.fori_loop(0, n // 2, loop_body, 0)

    gather_wait(ii0, ij0, xi0, xj0, sg0)


def kernel(particles, edges):
    table = particles.reshape(_N_NODES, 16)
    pad = _E_PAD - _E
    ei = jnp.concatenate([edges[0], jnp.zeros((pad,), jnp.int32)])
    ej = jnp.concatenate([edges[1], jnp.zeros((pad,), jnp.int32)])

    mesh = plsc.VectorSubcoreMesh(core_axis_name="c", subcore_axis_name="s")
    run = functools.partial(
        pl.kernel,
        mesh=mesh,
        compiler_params=pltpu.CompilerParams(
            use_tc_tiling_on_sc=False, needs_layout_passes=False),
        out_type=jax.ShapeDtypeStruct((_E_PAD, 16), jnp.float32),
        scratch_types=[
            pltpu.VMEM((_CHUNK,), jnp.int32),
            pltpu.VMEM((_CHUNK,), jnp.int32),
            pltpu.VMEM((_CHUNK,), jnp.int32),
            pltpu.VMEM((_CHUNK,), jnp.int32),
            pltpu.VMEM((_CHUNK, 16), jnp.float32),
            pltpu.VMEM((_CHUNK, 16), jnp.float32),
            pltpu.VMEM((_CHUNK, 16), jnp.float32),
            pltpu.VMEM((_CHUNK, 16), jnp.float32),
            pltpu.VMEM((_CHUNK, 16), jnp.float32),
            pltpu.SemaphoreType.DMA,
            pltpu.SemaphoreType.DMA,
        ],
    )(_sc_body)
    out = run(table, ei, ej)
    return out[:_E].reshape(_E, 4, 4)


# speedup vs baseline: 17.4611x; 1.3588x over previous
import functools

import jax
import jax.numpy as jnp
from jax import lax
from jax.experimental import pallas as pl
from jax.experimental.pallas import tpu as pltpu
from jax.experimental.pallas import tpu_sc as plsc

_N_NODES = 100000
_E = 1600000
_NC = 2
_NS = 16
_NW = _NC * _NS
_CHUNK = 256
_EPW = _E // _NW
_NCHUNKS = -(-_EPW // _CHUNK)
_NCHUNKS += _NCHUNKS % 2


def _rsqrt(x):
    i = plsc.bitcast(x, jnp.int32)
    i = jnp.int32(0x5F3759DF) - (i >> 1)
    y = plsc.bitcast(i, jnp.float32)
    xh = x * 0.5
    for _ in range(3):
        y = y * (1.5 - xh * y * y)
    return y


def _compute_chunk(xi_b, xj_b, ob):
    def block(b, carry):
        rows = lax.iota(jnp.int32, 16) + b * 16
        qi = [plsc.load_gather(xi_b, [rows, jnp.full((16,), c, jnp.int32)])
              for c in range(16)]
        qj = [plsc.load_gather(xj_b, [rows, jnp.full((16,), c, jnp.int32)])
              for c in range(16)]
        for p in range(4):
            aw, ax, ay, az = qi[4 * p:4 * p + 4]
            bw, bx, by, bz = qj[4 * p:4 * p + 4]
            w = aw * bw + ax * bx + ay * by + az * bz
            x = ax * bw - aw * bx - ay * bz + az * by
            y = ay * bw - aw * by + ax * bz - az * bx
            z = az * bw - aw * bz - ax * by + ay * bx
            r = _rsqrt(w * w + x * x + y * y + z * z)
            for c, v in enumerate((w * r, x * r, y * r, z * r)):
                col = jnp.full((16,), 4 * p + c, jnp.int32)
                plsc.store_scatter(ob, [rows, col], v)
        return carry

    lax.fori_loop(0, _CHUNK // 16, block, 0, unroll=4)


def _sc_body(table_hbm, edges_hbm, out_hbm,
             ii0, ij0, ii1, ij1, xi0, xj0, xi1, xj1, ob,
             sg0, sg1):
    wid = lax.axis_index("s") * _NC + lax.axis_index("c")
    wbase = wid * _EPW
    n = _NCHUNKS

    def idx_load(slot_ii, slot_ij, base):
        pltpu.sync_copy(edges_hbm.at[0, pl.ds(base, _CHUNK)], slot_ii)
        pltpu.sync_copy(edges_hbm.at[1, pl.ds(base, _CHUNK)], slot_ij)

    def gather_issue(slot_ii, slot_ij, xi_b, xj_b, sem):
        pltpu.async_copy(table_hbm.at[slot_ii], xi_b, sem)
        pltpu.async_copy(table_hbm.at[slot_ij], xj_b, sem)

    def gather_wait(slot_ii, slot_ij, xi_b, xj_b, sem):
        pltpu.make_async_copy(table_hbm.at[slot_ii], xi_b, sem).wait()
        pltpu.make_async_copy(table_hbm.at[slot_ij], xj_b, sem).wait()

    idx_load(ii0, ij0, wbase)
    gather_issue(ii0, ij0, xi0, xj0, sg0)

    def half_iter(g, cur, nxt):
        (ii_c, ij_c, xi_c, xj_c, sg_c) = cur
        (ii_n, ij_n, xi_n, xj_n, sg_n) = nxt
        base_g = wbase + jnp.minimum(g * _CHUNK, _EPW - _CHUNK)
        base_n = wbase + jnp.minimum((g + 1) * _CHUNK, _EPW - _CHUNK)
        idx_load(ii_n, ij_n, base_n)
        gather_issue(ii_n, ij_n, xi_n, xj_n, sg_n)
        gather_wait(ii_c, ij_c, xi_c, xj_c, sg_c)
        _compute_chunk(xi_c, xj_c, ob)
        pltpu.sync_copy(ob, out_hbm.at[pl.ds(base_g, _CHUNK)])

    slot0 = (ii0, ij0, xi0, xj0, sg0)
    slot1 = (ii1, ij1, xi1, xj1, sg1)

    def loop_body(t, carry):
        half_iter(2 * t, slot0, slot1)
        half_iter(2 * t + 1, slot1, slot0)
        return carry

    lax.fori_loop(0, n // 2, loop_body, 0)

    gather_wait(ii0, ij0, xi0, xj0, sg0)


def kernel(particles, edges):
    table = particles.reshape(_N_NODES, 16)

    mesh = plsc.VectorSubcoreMesh(core_axis_name="c", subcore_axis_name="s")
    run = functools.partial(
        pl.kernel,
        mesh=mesh,
        compiler_params=pltpu.CompilerParams(
            use_tc_tiling_on_sc=False, needs_layout_passes=False),
        out_type=jax.ShapeDtypeStruct((_E, 16), jnp.float32),
        scratch_types=[
            pltpu.VMEM((_CHUNK,), jnp.int32),
            pltpu.VMEM((_CHUNK,), jnp.int32),
            pltpu.VMEM((_CHUNK,), jnp.int32),
            pltpu.VMEM((_CHUNK,), jnp.int32),
            pltpu.VMEM((_CHUNK, 16), jnp.float32),
            pltpu.VMEM((_CHUNK, 16), jnp.float32),
            pltpu.VMEM((_CHUNK, 16), jnp.float32),
            pltpu.VMEM((_CHUNK, 16), jnp.float32),
            pltpu.VMEM((_CHUNK, 16), jnp.float32),
            pltpu.SemaphoreType.DMA,
            pltpu.SemaphoreType.DMA,
        ],
    )(_sc_body)
    out = run(table, edges)
    return out.reshape(_E, 4, 4)
